# baseline (speedup 1.0000x reference)
import functools

import jax
import jax.numpy as jnp
from jax import lax
from jax.experimental import pallas as pl
from jax.experimental.pallas import tpu as pltpu
from jax.experimental.pallas import tpu_sc as plsc

NC = 2
NS = 16
NW = NC * NS
L = 16
CN = 8
NSLOT = 3


def _sc_neighbor_sum(ne1, B, NB, D):
  nch = -(-B // (NW * CN))
  nch = -(-nch // NSLOT) * NSLOT
  npw = nch * CN

  mesh = plsc.VectorSubcoreMesh(
      core_axis_name="c", subcore_axis_name="s",
      num_cores=NC, num_subcores=NS)
  scratch = [
      [pltpu.VMEM((CN * NB * D,), jnp.float32)] * NSLOT,
      [pltpu.VMEM((CN * D,), jnp.float32)] * NSLOT,
      [pltpu.SemaphoreType.DMA] * NSLOT,
      [pltpu.SemaphoreType.DMA] * NSLOT,
  ]
  kd = D // L

  @functools.partial(
      pl.kernel,
      out_type=jax.ShapeDtypeStruct((B * D,), jnp.float32),
      mesh=mesh, scratch_types=scratch)
  def k(ne_h, ns_h, bne, one, sin, sout):
    wid = lax.axis_index("s") * NC + lax.axis_index("c")
    base = wid * npw

    def in_copy(ch, b):
      start = jnp.minimum(base + ch * CN, B - CN)
      return pltpu.make_async_copy(
          ne_h.at[pl.ds(start * NB * D, CN * NB * D)], bne[b], sin[b])

    def out_copy(ch, b):
      start = jnp.minimum(base + ch * CN, B - CN)
      return pltpu.make_async_copy(
          one[b], ns_h.at[pl.ds(start * D, CN * D)], sout[b])

    def compute(b):
      for i in range(CN):
        row0 = i * NB * D

        def nbody(n, accs):
          r = row0 + n * D
          return tuple(
              accs[d] + bne[b][pl.ds(r + d * L, L)] for d in range(kd))

        z = jnp.zeros((L,), jnp.float32)
        accs = lax.fori_loop(0, NB, nbody, (z,) * kd, unroll=4)
        for d in range(kd):
          one[b][pl.ds(i * D + d * L, L)] = accs[d]

    def slot(ch, b, t):
      in_copy(ch, b).wait()

      @pl.when(t > 0)
      def _():
        out_copy(ch - NSLOT, b).wait()

      compute(b)
      out_copy(ch, b).start()

      @pl.when(ch + NSLOT < nch)
      def _():
        in_copy(ch + NSLOT, b).start()

    for b in range(NSLOT):
      in_copy(b, b).start()

    def ring(t, carry):
      for b in range(NSLOT):
        slot(NSLOT * t + b, b, t)
      return carry

    lax.fori_loop(0, nch // NSLOT, ring, 0)
    for b in range(NSLOT):
      out_copy(nch - NSLOT + b, b).wait()

  return k(ne1)


def _tc_ef_partial(eft, W1c, B, NB, DE, D):
  nb_blk = 8

  def body(x_r, w_r, out_r):
    i = pl.program_id(0)
    s = jnp.sum(x_r[...], axis=0)
    part = lax.dot_general(s, w_r[...], (((0,), (0,)), ((), ())),
                           preferred_element_type=jnp.float32)

    @pl.when(i == 0)
    def _():
      out_r[...] = part

    @pl.when(i > 0)
    def _():
      out_r[...] = out_r[...] + part

  return pl.pallas_call(
      body,
      grid=(NB // nb_blk,),
      in_specs=[
          pl.BlockSpec((nb_blk, DE, B), lambda i: (i, 0, 0)),
          pl.BlockSpec((DE, D), lambda i: (0, 0)),
      ],
      out_specs=pl.BlockSpec((B, D), lambda i: (0, 0)),
      out_shape=jax.ShapeDtypeStruct((B, D), jnp.float32),
  )(eft, W1c)


def _tc_edge_partial(et2, P0, src, tm, W1b, b1, W2bc, b2, B, NB, DT, D, tb):
  def body(et_r, p0_r, src_r, tm_r, w1_r, b1_r, w2_r, b2_r, p_r, q_r):
    f32 = jnp.float32
    ts = jnp.sum(et_r[...].reshape(tb, NB, DT), axis=1)
    acc = jnp.dot(ts, w1_r[...], preferred_element_type=f32)
    p_r[...] = acc + p0_r[...] + f32(NB) * b1_r[0, :][None, :]
    q = jnp.dot(src_r[...], w2_r[0:D, :], preferred_element_type=f32)
    q = q + jnp.dot(tm_r[...], w2_r[D:2 * D, :], preferred_element_type=f32)
    q_r[...] = q + b2_r[0, :][None, :]

  return pl.pallas_call(
      body,
      grid=(B // tb,),
      in_specs=[
          pl.BlockSpec((tb * NB, DT), lambda i: (i, 0)),
          pl.BlockSpec((tb, D), lambda i: (i, 0)),
          pl.BlockSpec((tb, D), lambda i: (i, 0)),
          pl.BlockSpec((tb, DT), lambda i: (i, 0)),
          pl.BlockSpec((DT, D), lambda i: (0, 0)),
          pl.BlockSpec((1, D), lambda i: (0, 0)),
          pl.BlockSpec((2 * D, D), lambda i: (0, 0)),
          pl.BlockSpec((1, D), lambda i: (0, 0)),
      ],
      out_specs=[
          pl.BlockSpec((tb, D), lambda i: (i, 0)),
          pl.BlockSpec((tb, D), lambda i: (i, 0)),
      ],
      out_shape=[
          jax.ShapeDtypeStruct((B, D), jnp.float32),
          jax.ShapeDtypeStruct((B, D), jnp.float32),
      ],
  )(et2, P0, src, tm, W1b, b1, W2bc, b2)


def _tc_head(ns, P, Q, W1a, W2a, B, D, tb):
  def body(ns_r, p_r, q_r, w1_r, w2_r, out_r):
    f32 = jnp.float32
    h = jnp.maximum(
        jnp.dot(ns_r[...], w1_r[...], preferred_element_type=f32) + p_r[...],
        0.0)
    out_r[...] = jnp.dot(h, w2_r[...], preferred_element_type=f32) + q_r[...]

  return pl.pallas_call(
      body,
      grid=(B // tb,),
      in_specs=[
          pl.BlockSpec((tb, D), lambda i: (i, 0)),
          pl.BlockSpec((tb, D), lambda i: (i, 0)),
          pl.BlockSpec((tb, D), lambda i: (i, 0)),
          pl.BlockSpec((D, D), lambda i: (0, 0)),
          pl.BlockSpec((D, D), lambda i: (0, 0)),
      ],
      out_specs=pl.BlockSpec((tb, D), lambda i: (i, 0)),
      out_shape=jax.ShapeDtypeStruct((B, D), jnp.float32),
  )(ns, P, Q, W1a, W2a)


def kernel(n_layer, source_node_features, source_nodes_time_embedding,
           neighbor_embeddings, edge_time_embeddings, edge_features, mask,
           W1, b1, W2, b2):
  B, NB, D = neighbor_embeddings.shape
  DT = edge_time_embeddings.shape[2]
  DE = edge_features.shape[2]

  et2 = edge_time_embeddings.reshape(B * NB, DT)
  eft = jnp.transpose(edge_features, (1, 2, 0))

  ns = _sc_neighbor_sum(neighbor_embeddings.reshape(B * NB * D), B, NB,
                        D).reshape(B, D)

  src = source_node_features
  tm = jnp.squeeze(source_nodes_time_embedding, axis=1)
  P0 = _tc_ef_partial(eft, W1[D + DT:, :], B, NB, DE, D)
  P, Q = _tc_edge_partial(et2, P0, src, tm, W1[D:D + DT, :],
                          b1.reshape(1, D), W2[D:, :], b2.reshape(1, D),
                          B, NB, DT, D, tb=400)

  return _tc_head(ns, P, Q, W1[0:D, :], W2[0:D, :], B, D, tb=2000)


# speedup vs baseline: 1.0747x; 1.0747x over previous
import functools

import jax
import jax.numpy as jnp
from jax import lax
from jax.experimental import pallas as pl
from jax.experimental.pallas import tpu as pltpu
from jax.experimental.pallas import tpu_sc as plsc

NC = 2
NS = 16
NW = NC * NS
L = 16
CN = 8
NSLOT = 2


def _sc_neighbor_sum(ne1, B, NB, D):
  nch = -(-B // (NW * CN))
  nch = -(-nch // NSLOT) * NSLOT
  npw = nch * CN

  mesh = plsc.VectorSubcoreMesh(
      core_axis_name="c", subcore_axis_name="s",
      num_cores=NC, num_subcores=NS)
  scratch = [
      [pltpu.VMEM((CN * NB * D,), jnp.float32)] * NSLOT,
      [pltpu.VMEM((CN * D,), jnp.float32)] * NSLOT,
      [pltpu.SemaphoreType.DMA] * NSLOT,
      [pltpu.SemaphoreType.DMA] * NSLOT,
  ]
  kd = D // L

  @functools.partial(
      pl.kernel,
      out_type=jax.ShapeDtypeStruct((B * D,), jnp.float32),
      mesh=mesh, scratch_types=scratch)
  def k(ne_h, ns_h, bne, one, sin, sout):
    wid = lax.axis_index("s") * NC + lax.axis_index("c")
    base = wid * npw

    def in_copy(ch, b):
      start = jnp.minimum(base + ch * CN, B - CN)
      return pltpu.make_async_copy(
          ne_h.at[pl.ds(start * NB * D, CN * NB * D)], bne[b], sin[b])

    def out_copy(ch, b):
      start = jnp.minimum(base + ch * CN, B - CN)
      return pltpu.make_async_copy(
          one[b], ns_h.at[pl.ds(start * D, CN * D)], sout[b])

    def compute(b):
      for i in range(CN):
        row0 = i * NB * D

        def nbody(n, accs):
          r = row0 + n * D
          return tuple(
              accs[d] + bne[b][pl.ds(r + d * L, L)] for d in range(kd))

        z = jnp.zeros((L,), jnp.float32)
        accs = lax.fori_loop(0, NB, nbody, (z,) * kd, unroll=4)
        for d in range(kd):
          one[b][pl.ds(i * D + d * L, L)] = accs[d]

    def slot(ch, b, t):
      in_copy(ch, b).wait()

      @pl.when(t > 0)
      def _():
        out_copy(ch - NSLOT, b).wait()

      compute(b)
      out_copy(ch, b).start()

      @pl.when(ch + NSLOT < nch)
      def _():
        in_copy(ch + NSLOT, b).start()

    for b in range(NSLOT):
      in_copy(b, b).start()

    def ring(t, carry):
      for b in range(NSLOT):
        slot(NSLOT * t + b, b, t)
      return carry

    lax.fori_loop(0, nch // NSLOT, ring, 0)
    for b in range(NSLOT):
      out_copy(nch - NSLOT + b, b).wait()

  return k(ne1)


def _tc_ef_partial(eft, W1c, B, NB, DE, D):
  nb_blk = 8

  def body(x_r, w_r, out_r):
    i = pl.program_id(0)
    s = jnp.sum(x_r[...], axis=0)
    part = lax.dot_general(s, w_r[...], (((0,), (0,)), ((), ())),
                           preferred_element_type=jnp.float32)

    @pl.when(i == 0)
    def _():
      out_r[...] = part

    @pl.when(i > 0)
    def _():
      out_r[...] = out_r[...] + part

  return pl.pallas_call(
      body,
      grid=(NB // nb_blk,),
      in_specs=[
          pl.BlockSpec((nb_blk, DE, B), lambda i: (i, 0, 0)),
          pl.BlockSpec((DE, D), lambda i: (0, 0)),
      ],
      out_specs=pl.BlockSpec((B, D), lambda i: (0, 0)),
      out_shape=jax.ShapeDtypeStruct((B, D), jnp.float32),
  )(eft, W1c)


def _tc_edge_partial(et2, P0, src, tm, W1b, b1, W2bc, b2, B, NB, DT, D, tb):
  def body(et_r, p0_r, src_r, tm_r, w1_r, b1_r, w2_r, b2_r, p_r, q_r):
    f32 = jnp.float32
    ts = jnp.sum(et_r[...].reshape(tb, NB, DT), axis=1)
    acc = jnp.dot(ts, w1_r[...], preferred_element_type=f32)
    p_r[...] = acc + p0_r[...] + f32(NB) * b1_r[0, :][None, :]
    q = jnp.dot(src_r[...], w2_r[0:D, :], preferred_element_type=f32)
    q = q + jnp.dot(tm_r[...], w2_r[D:2 * D, :], preferred_element_type=f32)
    q_r[...] = q + b2_r[0, :][None, :]

  return pl.pallas_call(
      body,
      grid=(B // tb,),
      in_specs=[
          pl.BlockSpec((tb * NB, DT), lambda i: (i, 0)),
          pl.BlockSpec((tb, D), lambda i: (i, 0)),
          pl.BlockSpec((tb, D), lambda i: (i, 0)),
          pl.BlockSpec((tb, DT), lambda i: (i, 0)),
          pl.BlockSpec((DT, D), lambda i: (0, 0)),
          pl.BlockSpec((1, D), lambda i: (0, 0)),
          pl.BlockSpec((2 * D, D), lambda i: (0, 0)),
          pl.BlockSpec((1, D), lambda i: (0, 0)),
      ],
      out_specs=[
          pl.BlockSpec((tb, D), lambda i: (i, 0)),
          pl.BlockSpec((tb, D), lambda i: (i, 0)),
      ],
      out_shape=[
          jax.ShapeDtypeStruct((B, D), jnp.float32),
          jax.ShapeDtypeStruct((B, D), jnp.float32),
      ],
  )(et2, P0, src, tm, W1b, b1, W2bc, b2)


def _tc_head(ns, P, Q, W1a, W2a, B, D, tb):
  def body(ns_r, p_r, q_r, w1_r, w2_r, out_r):
    f32 = jnp.float32
    h = jnp.maximum(
        jnp.dot(ns_r[...], w1_r[...], preferred_element_type=f32) + p_r[...],
        0.0)
    out_r[...] = jnp.dot(h, w2_r[...], preferred_element_type=f32) + q_r[...]

  return pl.pallas_call(
      body,
      grid=(B // tb,),
      in_specs=[
          pl.BlockSpec((tb, D), lambda i: (i, 0)),
          pl.BlockSpec((tb, D), lambda i: (i, 0)),
          pl.BlockSpec((tb, D), lambda i: (i, 0)),
          pl.BlockSpec((D, D), lambda i: (0, 0)),
          pl.BlockSpec((D, D), lambda i: (0, 0)),
      ],
      out_specs=pl.BlockSpec((tb, D), lambda i: (i, 0)),
      out_shape=jax.ShapeDtypeStruct((B, D), jnp.float32),
  )(ns, P, Q, W1a, W2a)


def kernel(n_layer, source_node_features, source_nodes_time_embedding,
           neighbor_embeddings, edge_time_embeddings, edge_features, mask,
           W1, b1, W2, b2):
  B, NB, D = neighbor_embeddings.shape
  DT = edge_time_embeddings.shape[2]
  DE = edge_features.shape[2]

  et2 = edge_time_embeddings.reshape(B * NB, DT)
  eft = jnp.transpose(edge_features, (1, 2, 0))

  ns = _sc_neighbor_sum(neighbor_embeddings.reshape(B * NB * D), B, NB,
                        D).reshape(B, D)

  src = source_node_features
  tm = jnp.squeeze(source_nodes_time_embedding, axis=1)
  P0 = _tc_ef_partial(eft, W1[D + DT:, :], B, NB, DE, D)
  P, Q = _tc_edge_partial(et2, P0, src, tm, W1[D:D + DT, :],
                          b1.reshape(1, D), W2[D:, :], b2.reshape(1, D),
                          B, NB, DT, D, tb=1000)

  return _tc_head(ns, P, Q, W1[0:D, :], W2[0:D, :], B, D, tb=2000)


# speedup vs baseline: 1.0808x; 1.0056x over previous
import functools

import jax
import jax.numpy as jnp
from jax import lax
from jax.experimental import pallas as pl
from jax.experimental.pallas import tpu as pltpu
from jax.experimental.pallas import tpu_sc as plsc

NC = 2
NS = 16
NW = NC * NS
L = 16
CN = 8
NSLOT = 2


def _sc_neighbor_sum(ne1, B, NB, D):
  nch = -(-B // (NW * CN))
  nch = -(-nch // NSLOT) * NSLOT
  npw = nch * CN

  mesh = plsc.VectorSubcoreMesh(
      core_axis_name="c", subcore_axis_name="s",
      num_cores=NC, num_subcores=NS)
  scratch = [
      [pltpu.VMEM((CN * NB * D,), jnp.float32)] * NSLOT,
      [pltpu.VMEM((CN * D,), jnp.float32)] * NSLOT,
      [pltpu.SemaphoreType.DMA] * NSLOT,
      [pltpu.SemaphoreType.DMA] * NSLOT,
  ]
  kd = D // L

  @functools.partial(
      pl.kernel,
      out_type=jax.ShapeDtypeStruct((B * D,), jnp.float32),
      mesh=mesh, scratch_types=scratch)
  def k(ne_h, ns_h, bne, one, sin, sout):
    wid = lax.axis_index("s") * NC + lax.axis_index("c")
    base = wid * npw

    def in_copy(ch, b):
      start = jnp.minimum(base + ch * CN, B - CN)
      return pltpu.make_async_copy(
          ne_h.at[pl.ds(start * NB * D, CN * NB * D)], bne[b], sin[b])

    def out_copy(ch, b):
      start = jnp.minimum(base + ch * CN, B - CN)
      return pltpu.make_async_copy(
          one[b], ns_h.at[pl.ds(start * D, CN * D)], sout[b])

    def compute(b):
      for i in range(CN):
        row0 = i * NB * D

        def nbody(n, accs):
          r = row0 + n * D
          return tuple(
              accs[d] + bne[b][pl.ds(r + d * L, L)] for d in range(kd))

        z = jnp.zeros((L,), jnp.float32)
        accs = lax.fori_loop(0, NB, nbody, (z,) * kd, unroll=2)
        for d in range(kd):
          one[b][pl.ds(i * D + d * L, L)] = accs[d]

    def slot(ch, b, t):
      in_copy(ch, b).wait()

      @pl.when(t > 0)
      def _():
        out_copy(ch - NSLOT, b).wait()

      compute(b)
      out_copy(ch, b).start()

      @pl.when(ch + NSLOT < nch)
      def _():
        in_copy(ch + NSLOT, b).start()

    for b in range(NSLOT):
      in_copy(b, b).start()

    def ring(t, carry):
      for b in range(NSLOT):
        slot(NSLOT * t + b, b, t)
      return carry

    lax.fori_loop(0, nch // NSLOT, ring, 0)
    for b in range(NSLOT):
      out_copy(nch - NSLOT + b, b).wait()

  return k(ne1)


def _tc_ef_partial(eft, W1c, B, NB, DE, D):
  nb_blk = 8

  def body(x_r, w_r, out_r):
    i = pl.program_id(0)
    s = jnp.sum(x_r[...], axis=0)
    part = lax.dot_general(s, w_r[...], (((0,), (0,)), ((), ())),
                           preferred_element_type=jnp.float32)

    @pl.when(i == 0)
    def _():
      out_r[...] = part

    @pl.when(i > 0)
    def _():
      out_r[...] = out_r[...] + part

  return pl.pallas_call(
      body,
      grid=(NB // nb_blk,),
      in_specs=[
          pl.BlockSpec((nb_blk, DE, B), lambda i: (i, 0, 0)),
          pl.BlockSpec((DE, D), lambda i: (0, 0)),
      ],
      out_specs=pl.BlockSpec((B, D), lambda i: (0, 0)),
      out_shape=jax.ShapeDtypeStruct((B, D), jnp.float32),
  )(eft, W1c)


def _tc_edge_partial(et2, P0, src, tm, W1b, b1, W2bc, b2, B, NB, DT, D, tb):
  def body(et_r, p0_r, src_r, tm_r, w1_r, b1_r, w2_r, b2_r, p_r, q_r):
    f32 = jnp.float32
    ts = jnp.sum(et_r[...].reshape(tb, NB, DT), axis=1)
    acc = jnp.dot(ts, w1_r[...], preferred_element_type=f32)
    p_r[...] = acc + p0_r[...] + f32(NB) * b1_r[0, :][None, :]
    q = jnp.dot(src_r[...], w2_r[0:D, :], preferred_element_type=f32)
    q = q + jnp.dot(tm_r[...], w2_r[D:2 * D, :], preferred_element_type=f32)
    q_r[...] = q + b2_r[0, :][None, :]

  return pl.pallas_call(
      body,
      grid=(B // tb,),
      in_specs=[
          pl.BlockSpec((tb * NB, DT), lambda i: (i, 0)),
          pl.BlockSpec((tb, D), lambda i: (i, 0)),
          pl.BlockSpec((tb, D), lambda i: (i, 0)),
          pl.BlockSpec((tb, DT), lambda i: (i, 0)),
          pl.BlockSpec((DT, D), lambda i: (0, 0)),
          pl.BlockSpec((1, D), lambda i: (0, 0)),
          pl.BlockSpec((2 * D, D), lambda i: (0, 0)),
          pl.BlockSpec((1, D), lambda i: (0, 0)),
      ],
      out_specs=[
          pl.BlockSpec((tb, D), lambda i: (i, 0)),
          pl.BlockSpec((tb, D), lambda i: (i, 0)),
      ],
      out_shape=[
          jax.ShapeDtypeStruct((B, D), jnp.float32),
          jax.ShapeDtypeStruct((B, D), jnp.float32),
      ],
  )(et2, P0, src, tm, W1b, b1, W2bc, b2)


def _tc_head(ns, P, Q, W1a, W2a, B, D, tb):
  def body(ns_r, p_r, q_r, w1_r, w2_r, out_r):
    f32 = jnp.float32
    h = jnp.maximum(
        jnp.dot(ns_r[...], w1_r[...], preferred_element_type=f32) + p_r[...],
        0.0)
    out_r[...] = jnp.dot(h, w2_r[...], preferred_element_type=f32) + q_r[...]

  return pl.pallas_call(
      body,
      grid=(B // tb,),
      in_specs=[
          pl.BlockSpec((tb, D), lambda i: (i, 0)),
          pl.BlockSpec((tb, D), lambda i: (i, 0)),
          pl.BlockSpec((tb, D), lambda i: (i, 0)),
          pl.BlockSpec((D, D), lambda i: (0, 0)),
          pl.BlockSpec((D, D), lambda i: (0, 0)),
      ],
      out_specs=pl.BlockSpec((tb, D), lambda i: (i, 0)),
      out_shape=jax.ShapeDtypeStruct((B, D), jnp.float32),
  )(ns, P, Q, W1a, W2a)


def kernel(n_layer, source_node_features, source_nodes_time_embedding,
           neighbor_embeddings, edge_time_embeddings, edge_features, mask,
           W1, b1, W2, b2):
  B, NB, D = neighbor_embeddings.shape
  DT = edge_time_embeddings.shape[2]
  DE = edge_features.shape[2]

  et2 = edge_time_embeddings.reshape(B * NB, DT)
  eft = jnp.transpose(edge_features, (1, 2, 0))

  ns = _sc_neighbor_sum(neighbor_embeddings.reshape(B * NB * D), B, NB,
                        D).reshape(B, D)

  src = source_node_features
  tm = jnp.squeeze(source_nodes_time_embedding, axis=1)
  P0 = _tc_ef_partial(eft, W1[D + DT:, :], B, NB, DE, D)
  P, Q = _tc_edge_partial(et2, P0, src, tm, W1[D:D + DT, :],
                          b1.reshape(1, D), W2[D:, :], b2.reshape(1, D),
                          B, NB, DT, D, tb=400)

  return _tc_head(ns, P, Q, W1[0:D, :], W2[0:D, :], B, D, tb=2000)
